# baseline (speedup 1.0000x reference)
import functools

import jax
import jax.numpy as jnp
from jax import lax
from jax.experimental import pallas as pl
from jax.experimental.pallas import tpu as pltpu
from jax.experimental.pallas import tpu_sc as plsc

N_NODES = 10000
NPAD = 10240
D = 256
H = 128
E = 160000
CHUNK = 64
GC = 8
NBUF = 4
EPAD = 163840
CHUNKS_PER_TILE = EPAD // 16 // CHUNK
GROUPS_PER_TILE = CHUNKS_PER_TILE // GC
EDGES_PER_GROUP = GC * CHUNK
ROWS_PER_TILE = NPAD // 16
NT = 1024
NTT = NT // CHUNK
TPT = CHUNK

_mesh = plsc.VectorSubcoreMesh(core_axis_name="c", subcore_axis_name="s")
_sc_params = pltpu.CompilerParams(needs_layout_passes=False)


def _edge_sweep(x_hbm, src_hbm, dst_hbm, zrows_hbm, s, srcg, dstg, rows,
                acc, gsem, ssem, degloc=None):
    ones16 = jnp.ones((16,), jnp.float32)
    base = s * GROUPS_PER_TILE

    pltpu.sync_copy(src_hbm.at[pl.ds(base * GC, GC)], srcg)
    pltpu.sync_copy(dst_hbm.at[pl.ds(base * GC, GC)], dstg)
    for p in range(NBUF):
        pltpu.sync_copy(zrows_hbm.at[pl.ds(0, CHUNK)], rows[p])
    for p in range(NBUF):
        pltpu.async_copy(rows[p], acc.at[dstg.at[0]], ssem[p], add=True)

    def group(g, carry):
        @pl.when(g > 0)
        def _stage():
            pltpu.sync_copy(src_hbm.at[pl.ds((base + g) * GC, GC)], srcg)
            pltpu.sync_copy(dst_hbm.at[pl.ds((base + g) * GC, GC)], dstg)
        for b in range(2):
            pltpu.make_async_copy(rows[b], acc.at[dstg.at[0]], ssem[b]).wait()
            pltpu.async_copy(x_hbm.at[srcg.at[b]], rows[b], gsem[b])
        for b in range(GC):
            p = b % NBUF
            pltpu.make_async_copy(x_hbm.at[srcg.at[b]], rows[p], gsem[p]).wait()
            if degloc is not None:
                for k in range(CHUNK // 16):
                    idx16 = dstg[b, pl.ds(k * 16, 16)]
                    plsc.addupdate_scatter(degloc, [idx16], ones16)
            pltpu.async_copy(rows[p], acc.at[dstg.at[b]], ssem[p], add=True)
            if b + 2 < GC:
                q = (b + 2) % NBUF
                pltpu.make_async_copy(
                    rows[q], acc.at[dstg.at[0]], ssem[q]).wait()
                pltpu.async_copy(x_hbm.at[srcg.at[b + 2]], rows[q], gsem[q])
        return carry

    lax.fori_loop(0, GROUPS_PER_TILE, group, 0)
    for p in range(NBUF):
        pltpu.make_async_copy(rows[p], acc.at[dstg.at[0]], ssem[p]).wait()


def _segsum0_body(xa_hbm, xb_hbm, src_hbm, dst_hbm, zrows_hbm, zvec_hbm,
                  agg_hbm, degp_hbm, srcg, dstg,
                  rows0, rows1, rows2, rows3, degloc, acc,
                  gsem0, gsem1, gsem2, gsem3, ssem0, ssem1, ssem2, ssem3):
    c = lax.axis_index("c")
    s = lax.axis_index("s")
    rows = (rows0, rows1, rows2, rows3)
    gsem = (gsem0, gsem1, gsem2, gsem3)
    ssem = (ssem0, ssem1, ssem2, ssem3)
    pltpu.sync_copy(zrows_hbm, acc.at[pl.ds(s * ROWS_PER_TILE, ROWS_PER_TILE)])
    pltpu.sync_copy(zvec_hbm, degloc)
    plsc.subcore_barrier()

    pl.when(c == 0)(lambda: _edge_sweep(
        xa_hbm, src_hbm, dst_hbm, zrows_hbm, s, srcg, dstg, rows, acc,
        gsem, ssem, degloc))
    pl.when(c == 1)(lambda: _edge_sweep(
        xb_hbm, src_hbm, dst_hbm, zrows_hbm, s, srcg, dstg, rows, acc,
        gsem, ssem, degloc))
    pltpu.sync_copy(degloc, degp_hbm.at[c, s])
    plsc.subcore_barrier()
    pltpu.sync_copy(acc.at[pl.ds(s * ROWS_PER_TILE, ROWS_PER_TILE)],
                    agg_hbm.at[c, pl.ds(s * ROWS_PER_TILE, ROWS_PER_TILE)])


_seg0 = functools.partial(
    pl.kernel,
    mesh=_mesh,
    out_type=[
        jax.ShapeDtypeStruct((2, NPAD, H), jnp.float32),
        jax.ShapeDtypeStruct((2, 16, NPAD), jnp.float32),
    ],
    scratch_types=[
        pltpu.VMEM((GC, CHUNK), jnp.int32),
        pltpu.VMEM((GC, CHUNK), jnp.int32),
        pltpu.VMEM((CHUNK, H), jnp.float32),
        pltpu.VMEM((CHUNK, H), jnp.float32),
        pltpu.VMEM((CHUNK, H), jnp.float32),
        pltpu.VMEM((CHUNK, H), jnp.float32),
        pltpu.VMEM((NPAD,), jnp.float32),
        pltpu.VMEM_SHARED((NPAD, H), jnp.float32),
        pltpu.SemaphoreType.DMA,
        pltpu.SemaphoreType.DMA,
        pltpu.SemaphoreType.DMA,
        pltpu.SemaphoreType.DMA,
        pltpu.SemaphoreType.DMA,
        pltpu.SemaphoreType.DMA,
        pltpu.SemaphoreType.DMA,
        pltpu.SemaphoreType.DMA,
    ],
    compiler_params=_sc_params,
)(_segsum0_body)


def _segsum1_body(ha_hbm, hb_hbm, src_hbm, dst_hbm, ti_hbm, dinv_hbm,
                  zrows_hbm, aggt_hbm, ht_hbm,
                  srcg, dstg, tiv, rows0, rows1, rows2, rows3, dinvv, acc,
                  gsem0, gsem1, gsem2, gsem3, ssem0, ssem1, ssem2, ssem3):
    c = lax.axis_index("c")
    s = lax.axis_index("s")
    rows = (rows0, rows1, rows2, rows3)
    gsem = (gsem0, gsem1, gsem2, gsem3)
    ssem = (ssem0, ssem1, ssem2, ssem3)
    pltpu.sync_copy(zrows_hbm, acc.at[pl.ds(s * ROWS_PER_TILE, ROWS_PER_TILE)])
    pl.when(s < NTT)(
        lambda: pltpu.sync_copy(ti_hbm.at[pl.ds(s * TPT, TPT)], tiv))
    pltpu.sync_copy(dinv_hbm, dinvv)
    plsc.subcore_barrier()

    def run(h_hbm):
        _edge_sweep(h_hbm, src_hbm, dst_hbm, zrows_hbm, s, srcg, dstg, rows,
                    acc, gsem, ssem)
        pl.when(s < NTT)(
            lambda: pltpu.async_copy(h_hbm.at[tiv], rows1, gsem1).wait())

    pl.when(c == 0)(lambda: run(ha_hbm))
    pl.when(c == 1)(lambda: run(hb_hbm))
    plsc.subcore_barrier()

    def targets():
        pltpu.async_copy(acc.at[tiv], rows0, gsem0).wait()
        for g in range(TPT // 16):
            tiv16 = tiv[pl.ds(g * 16, 16)]
            dinv16 = plsc.load_gather(dinvv, [tiv16])
            for l in range(16):
                d = dinv16[l]
                r = g * 16 + l
                for k in range(H // 16):
                    rows0[r, pl.ds(k * 16, 16)] = (
                        rows0[r, pl.ds(k * 16, 16)] * d)
        pltpu.sync_copy(rows0, aggt_hbm.at[c, pl.ds(s * TPT, TPT)])
        pltpu.sync_copy(rows1, ht_hbm.at[c, pl.ds(s * TPT, TPT)])

    pl.when(s < NTT)(targets)


_seg1 = functools.partial(
    pl.kernel,
    mesh=_mesh,
    out_type=[
        jax.ShapeDtypeStruct((2, NT, H), jnp.float32),
        jax.ShapeDtypeStruct((2, NT, H), jnp.float32),
    ],
    scratch_types=[
        pltpu.VMEM((GC, CHUNK), jnp.int32),
        pltpu.VMEM((GC, CHUNK), jnp.int32),
        pltpu.VMEM((TPT,), jnp.int32),
        pltpu.VMEM((CHUNK, H), jnp.float32),
        pltpu.VMEM((CHUNK, H), jnp.float32),
        pltpu.VMEM((CHUNK, H), jnp.float32),
        pltpu.VMEM((CHUNK, H), jnp.float32),
        pltpu.VMEM((NPAD,), jnp.float32),
        pltpu.VMEM_SHARED((NPAD, H), jnp.float32),
        pltpu.SemaphoreType.DMA,
        pltpu.SemaphoreType.DMA,
        pltpu.SemaphoreType.DMA,
        pltpu.SemaphoreType.DMA,
        pltpu.SemaphoreType.DMA,
        pltpu.SemaphoreType.DMA,
        pltpu.SemaphoreType.DMA,
        pltpu.SemaphoreType.DMA,
    ],
    compiler_params=_sc_params,
)(_segsum1_body)


_HI = jax.lax.Precision.HIGHEST


def _layer0_body(x_ref, aa_ref, ab_ref, dp_ref, ws_ref, wn_ref, b_ref,
                 oa_ref, ob_ref, od_ref):
    deg = jnp.sum(dp_ref[0], axis=0)
    dinv = 1.0 / jnp.maximum(deg, 1.0)
    dcol = dinv[:, None]
    a = aa_ref[0] * dcol
    b = ab_ref[0] * dcol
    h = jax.lax.dot(x_ref[...], ws_ref[...], precision=_HI)
    h = h + jax.lax.dot(a, wn_ref[:H, :], precision=_HI)
    h = h + jax.lax.dot(b, wn_ref[H:, :], precision=_HI)
    h = jnp.maximum(h + b_ref[...], 0.0)
    oa_ref[...] = h[:, :H]
    ob_ref[...] = h[:, H:]
    od_ref[...] = dinv.reshape(od_ref.shape)


_R0 = 1024


def _layer0(xp, aggdeg, degp, W0_self, W0_neigh, b0):
    return pl.pallas_call(
        _layer0_body,
        grid=(NPAD // _R0,),
        in_specs=[
            pl.BlockSpec((_R0, D), lambda i: (i, 0)),
            pl.BlockSpec((1, _R0, H), lambda i: (0, i, 0)),
            pl.BlockSpec((1, _R0, H), lambda i: (1, i, 0)),
            pl.BlockSpec((1, 16, _R0), lambda i: (0, 0, i)),
            pl.BlockSpec((D, D), lambda i: (0, 0)),
            pl.BlockSpec((D, D), lambda i: (0, 0)),
            pl.BlockSpec((1, D), lambda i: (0, 0)),
        ],
        out_specs=[
            pl.BlockSpec((_R0, H), lambda i: (i, 0)),
            pl.BlockSpec((_R0, H), lambda i: (i, 0)),
            pl.BlockSpec((_R0 // 128, 128), lambda i: (i, 0)),
        ],
        out_shape=[
            jax.ShapeDtypeStruct((NPAD, H), jnp.float32),
            jax.ShapeDtypeStruct((NPAD, H), jnp.float32),
            jax.ShapeDtypeStruct((NPAD // 128, 128), jnp.float32),
        ],
    )(xp, aggdeg, aggdeg, degp, W0_self, W0_neigh, b0)


def _layer1_body(at_ref, ht_ref, ws_ref, wn_ref, b_ref, o_ref):
    hl = ht_ref[0]
    hh = ht_ref[1]
    o = jax.lax.dot(hl, ws_ref[:H, :], precision=_HI)
    o = o + jax.lax.dot(hh, ws_ref[H:, :], precision=_HI)
    o = o + jax.lax.dot(at_ref[0], wn_ref[:H, :], precision=_HI)
    o = o + jax.lax.dot(at_ref[1], wn_ref[H:, :], precision=_HI)
    o_ref[...] = o + b_ref[...]


def _layer1(aggt, ht, W1_self, W1_neigh, b1):
    return pl.pallas_call(
        _layer1_body,
        out_shape=jax.ShapeDtypeStruct((NT, D), jnp.float32),
    )(aggt, ht, W1_self, W1_neigh, b1)


def kernel(x, edge_index, target_indices, W0_self, W0_neigh, b0,
           W1_self, W1_neigh, b1):
    f32 = jnp.float32
    x = x.astype(f32)
    src = edge_index[0]
    dst = edge_index[1]

    xp = jnp.zeros((NPAD, D), f32).at[:N_NODES].set(x)
    xa = xp[:, :H]
    xb = xp[:, H:]

    pad = EPAD - E
    srcp = jnp.concatenate([src, jnp.zeros((pad,), jnp.int32)]).reshape(
        EPAD // CHUNK, CHUNK)
    dstp = jnp.concatenate([dst, jnp.full((pad,), N_NODES, jnp.int32)]).reshape(
        EPAD // CHUNK, CHUNK)
    zrows = jnp.zeros((ROWS_PER_TILE, H), f32)
    zvec = jnp.zeros((NPAD,), f32)

    agg_raw, degp = _seg0(xa, xb, srcp, dstp, zrows, zvec)
    h1a, h1b, dinv2 = _layer0(xp, agg_raw, degp, W0_self, W0_neigh,
                              b0.reshape(1, D))
    aggt, ht = _seg1(h1a, h1b, srcp, dstp, target_indices,
                     dinv2.reshape(NPAD), zrows)
    out = _layer1(aggt, ht, W1_self, W1_neigh, b1.reshape(1, D))
    return out


# speedup vs baseline: 1.3092x; 1.3092x over previous
import functools

import jax
import jax.numpy as jnp
from jax import lax
from jax.experimental import pallas as pl
from jax.experimental.pallas import tpu as pltpu
from jax.experimental.pallas import tpu_sc as plsc

N_NODES = 10000
NPAD = 10240
D = 256
H = 128
E = 160000
CHUNK = 64
GC = 8
NBUF = 4
EPAD = 163840
CHUNKS_PER_TILE = EPAD // 16 // CHUNK
GROUPS_PER_TILE = CHUNKS_PER_TILE // GC
EDGES_PER_GROUP = GC * CHUNK
ROWS_PER_TILE = NPAD // 16
NT = 1024
NTT = NT // CHUNK
TPT = CHUNK
DUMMY = N_NODES
EPG = GC * CHUNK
CAPT = 10752
CAPG = CAPT // EPG
CAPC = CAPT // CHUNK

_mesh = plsc.VectorSubcoreMesh(core_axis_name="c", subcore_axis_name="s")
_sc_params = pltpu.CompilerParams(needs_layout_passes=False)


def _edge_sweep(x_hbm, src_hbm, dst_hbm, zrows_hbm, chunk_base, n_groups,
                srcg, dstg, rows, acc, gsem, ssem, degloc=None):
    ones16 = jnp.ones((16,), jnp.float32)

    pltpu.sync_copy(src_hbm.at[pl.ds(chunk_base, GC)], srcg)
    pltpu.sync_copy(dst_hbm.at[pl.ds(chunk_base, GC)], dstg)
    for p in range(NBUF):
        pltpu.sync_copy(zrows_hbm.at[pl.ds(0, CHUNK)], rows[p])
    for p in range(NBUF):
        pltpu.async_copy(rows[p], acc.at[dstg.at[0]], ssem[p], add=True)

    def group(g, carry):
        @pl.when(g > 0)
        def _stage():
            pltpu.sync_copy(src_hbm.at[pl.ds(chunk_base + g * GC, GC)], srcg)
            pltpu.sync_copy(dst_hbm.at[pl.ds(chunk_base + g * GC, GC)], dstg)
        for b in range(2):
            pltpu.make_async_copy(rows[b], acc.at[dstg.at[0]], ssem[b]).wait()
            pltpu.async_copy(x_hbm.at[srcg.at[b]], rows[b], gsem[b])
        for b in range(GC):
            p = b % NBUF
            pltpu.make_async_copy(x_hbm.at[srcg.at[b]], rows[p], gsem[p]).wait()
            if degloc is not None:
                for k in range(CHUNK // 16):
                    idx16 = dstg[b, pl.ds(k * 16, 16)]
                    plsc.addupdate_scatter(degloc, [idx16], ones16)
            pltpu.async_copy(rows[p], acc.at[dstg.at[b]], ssem[p], add=True)
            if b + 2 < GC:
                q = (b + 2) % NBUF
                pltpu.make_async_copy(
                    rows[q], acc.at[dstg.at[0]], ssem[q]).wait()
                pltpu.async_copy(x_hbm.at[srcg.at[b + 2]], rows[q], gsem[q])
        return carry

    lax.fori_loop(0, n_groups, group, 0)
    for p in range(NBUF):
        pltpu.make_async_copy(rows[p], acc.at[dstg.at[0]], ssem[p]).wait()


def _filter_body(src_hbm, dst_hbm, ti_hbm, zvec_hbm, dummy_hbm,
                 srcf_hbm, dstf_hbm, cnt_hbm,
                 srcv, dstv, srcf, dstf, istgt, tivf, cntv):
    c = lax.axis_index("c")
    s = lax.axis_index("s")
    ones16 = jnp.ones((16,), jnp.float32)

    def run():
        pltpu.sync_copy(zvec_hbm, istgt)
        pltpu.sync_copy(ti_hbm, tivf)
        for g in range(NT // 16):
            t16 = tivf[pl.ds(g * 16, 16)]
            plsc.store_scatter(istgt, [t16], ones16)
        pltpu.sync_copy(src_hbm.at[pl.ds(s * CHUNKS_PER_TILE, CHUNKS_PER_TILE)],
                        srcv)
        pltpu.sync_copy(dst_hbm.at[pl.ds(s * CHUNKS_PER_TILE, CHUNKS_PER_TILE)],
                        dstv)
        pltpu.sync_copy(dummy_hbm, srcf)
        pltpu.sync_copy(dummy_hbm, dstf)

        def chunk(ch, off):
            for k in range(CHUNK // 16):
                d16 = dstv[ch, pl.ds(k * 16, 16)]
                s16 = srcv[ch, pl.ds(k * 16, 16)]
                m16 = plsc.load_gather(istgt, [d16]) > 0.5
                plsc.store_compressed(srcf.at[pl.ds(off, 16)], s16, mask=m16)
                plsc.store_compressed(dstf.at[pl.ds(off, 16)], d16, mask=m16)
                off = off + plsc.all_reduce_population_count(m16)[0]
            return off

        off = lax.fori_loop(0, CHUNKS_PER_TILE, chunk, jnp.int32(0))
        off_r = ((off + EPG - 1) // EPG) * EPG
        pltpu.sync_copy(srcf, srcf_hbm.at[s])
        pltpu.sync_copy(dstf, dstf_hbm.at[s])
        cntv[...] = jnp.full((16,), 0, jnp.int32) + off_r
        pltpu.sync_copy(cntv, cnt_hbm.at[s])

    pl.when(c == 0)(run)


_filt = functools.partial(
    pl.kernel,
    mesh=_mesh,
    out_type=[
        jax.ShapeDtypeStruct((16, CAPT), jnp.int32),
        jax.ShapeDtypeStruct((16, CAPT), jnp.int32),
        jax.ShapeDtypeStruct((16, 16), jnp.int32),
    ],
    scratch_types=[
        pltpu.VMEM((CHUNKS_PER_TILE, CHUNK), jnp.int32),
        pltpu.VMEM((CHUNKS_PER_TILE, CHUNK), jnp.int32),
        pltpu.VMEM((CAPT,), jnp.int32),
        pltpu.VMEM((CAPT,), jnp.int32),
        pltpu.VMEM((NPAD,), jnp.float32),
        pltpu.VMEM((NT,), jnp.int32),
        pltpu.VMEM((16,), jnp.int32),
    ],
    compiler_params=_sc_params,
)(_filter_body)


def _segsum0_body(xa_hbm, xb_hbm, src_hbm, dst_hbm, zrows_hbm, zvec_hbm,
                  agg_hbm, degp_hbm, srcg, dstg,
                  rows0, rows1, rows2, rows3, degloc, acc,
                  gsem0, gsem1, gsem2, gsem3, ssem0, ssem1, ssem2, ssem3):
    c = lax.axis_index("c")
    s = lax.axis_index("s")
    rows = (rows0, rows1, rows2, rows3)
    gsem = (gsem0, gsem1, gsem2, gsem3)
    ssem = (ssem0, ssem1, ssem2, ssem3)
    pltpu.sync_copy(zrows_hbm, acc.at[pl.ds(s * ROWS_PER_TILE, ROWS_PER_TILE)])
    pltpu.sync_copy(zvec_hbm, degloc)
    plsc.subcore_barrier()

    pl.when(c == 0)(lambda: _edge_sweep(
        xa_hbm, src_hbm, dst_hbm, zrows_hbm, s * CHUNKS_PER_TILE,
        GROUPS_PER_TILE, srcg, dstg, rows, acc, gsem, ssem, degloc))
    pl.when(c == 1)(lambda: _edge_sweep(
        xb_hbm, src_hbm, dst_hbm, zrows_hbm, s * CHUNKS_PER_TILE,
        GROUPS_PER_TILE, srcg, dstg, rows, acc, gsem, ssem, degloc))
    pltpu.sync_copy(degloc, degp_hbm.at[c, s])
    plsc.subcore_barrier()
    pltpu.sync_copy(acc.at[pl.ds(s * ROWS_PER_TILE, ROWS_PER_TILE)],
                    agg_hbm.at[c, pl.ds(s * ROWS_PER_TILE, ROWS_PER_TILE)])


_seg0 = functools.partial(
    pl.kernel,
    mesh=_mesh,
    out_type=[
        jax.ShapeDtypeStruct((2, NPAD, H), jnp.float32),
        jax.ShapeDtypeStruct((2, 16, NPAD), jnp.float32),
    ],
    scratch_types=[
        pltpu.VMEM((GC, CHUNK), jnp.int32),
        pltpu.VMEM((GC, CHUNK), jnp.int32),
        pltpu.VMEM((CHUNK, H), jnp.float32),
        pltpu.VMEM((CHUNK, H), jnp.float32),
        pltpu.VMEM((CHUNK, H), jnp.float32),
        pltpu.VMEM((CHUNK, H), jnp.float32),
        pltpu.VMEM((NPAD,), jnp.float32),
        pltpu.VMEM_SHARED((NPAD, H), jnp.float32),
        pltpu.SemaphoreType.DMA,
        pltpu.SemaphoreType.DMA,
        pltpu.SemaphoreType.DMA,
        pltpu.SemaphoreType.DMA,
        pltpu.SemaphoreType.DMA,
        pltpu.SemaphoreType.DMA,
        pltpu.SemaphoreType.DMA,
        pltpu.SemaphoreType.DMA,
    ],
    compiler_params=_sc_params,
)(_segsum0_body)


def _segsum1_body(ha_hbm, hb_hbm, src_hbm, dst_hbm, cnt_hbm, ti_hbm, dinv_hbm,
                  zrows_hbm, aggt_hbm, ht_hbm,
                  srcg, dstg, tiv, cntv, rows0, rows1, rows2, rows3, dinvv,
                  acc, gsem0, gsem1, gsem2, gsem3, ssem0, ssem1, ssem2, ssem3):
    c = lax.axis_index("c")
    s = lax.axis_index("s")
    rows = (rows0, rows1, rows2, rows3)
    gsem = (gsem0, gsem1, gsem2, gsem3)
    ssem = (ssem0, ssem1, ssem2, ssem3)
    pltpu.sync_copy(zrows_hbm, acc.at[pl.ds(s * ROWS_PER_TILE, ROWS_PER_TILE)])
    pl.when(s < NTT)(
        lambda: pltpu.sync_copy(ti_hbm.at[pl.ds(s * TPT, TPT)], tiv))
    pltpu.sync_copy(dinv_hbm, dinvv)
    pltpu.sync_copy(cnt_hbm.at[s], cntv)
    n_groups = cntv[...][0] // EPG
    plsc.subcore_barrier()

    def run(h_hbm):
        _edge_sweep(h_hbm, src_hbm, dst_hbm, zrows_hbm, s * CAPC, n_groups,
                    srcg, dstg, rows, acc, gsem, ssem)
        pl.when(s < NTT)(
            lambda: pltpu.async_copy(h_hbm.at[tiv], rows1, gsem1).wait())

    pl.when(c == 0)(lambda: run(ha_hbm))
    pl.when(c == 1)(lambda: run(hb_hbm))
    plsc.subcore_barrier()

    def targets():
        pltpu.async_copy(acc.at[tiv], rows0, gsem0).wait()
        for g in range(TPT // 16):
            tiv16 = tiv[pl.ds(g * 16, 16)]
            dinv16 = plsc.load_gather(dinvv, [tiv16])
            for l in range(16):
                d = dinv16[l]
                r = g * 16 + l
                for k in range(H // 16):
                    rows0[r, pl.ds(k * 16, 16)] = (
                        rows0[r, pl.ds(k * 16, 16)] * d)
        pltpu.sync_copy(rows0, aggt_hbm.at[c, pl.ds(s * TPT, TPT)])
        pltpu.sync_copy(rows1, ht_hbm.at[c, pl.ds(s * TPT, TPT)])

    pl.when(s < NTT)(targets)


_seg1 = functools.partial(
    pl.kernel,
    mesh=_mesh,
    out_type=[
        jax.ShapeDtypeStruct((2, NT, H), jnp.float32),
        jax.ShapeDtypeStruct((2, NT, H), jnp.float32),
    ],
    scratch_types=[
        pltpu.VMEM((GC, CHUNK), jnp.int32),
        pltpu.VMEM((GC, CHUNK), jnp.int32),
        pltpu.VMEM((TPT,), jnp.int32),
        pltpu.VMEM((16,), jnp.int32),
        pltpu.VMEM((CHUNK, H), jnp.float32),
        pltpu.VMEM((CHUNK, H), jnp.float32),
        pltpu.VMEM((CHUNK, H), jnp.float32),
        pltpu.VMEM((CHUNK, H), jnp.float32),
        pltpu.VMEM((NPAD,), jnp.float32),
        pltpu.VMEM_SHARED((NPAD, H), jnp.float32),
        pltpu.SemaphoreType.DMA,
        pltpu.SemaphoreType.DMA,
        pltpu.SemaphoreType.DMA,
        pltpu.SemaphoreType.DMA,
        pltpu.SemaphoreType.DMA,
        pltpu.SemaphoreType.DMA,
        pltpu.SemaphoreType.DMA,
        pltpu.SemaphoreType.DMA,
    ],
    compiler_params=_sc_params,
)(_segsum1_body)


_HI = jax.lax.Precision.HIGHEST


def _layer0_body(x_ref, aa_ref, ab_ref, dp_ref, ws_ref, wn_ref, b_ref,
                 oa_ref, ob_ref, od_ref):
    deg = jnp.sum(dp_ref[0], axis=0)
    dinv = 1.0 / jnp.maximum(deg, 1.0)
    dcol = dinv[:, None]
    a = aa_ref[0] * dcol
    b = ab_ref[0] * dcol
    h = jax.lax.dot(x_ref[...], ws_ref[...], precision=_HI)
    h = h + jax.lax.dot(a, wn_ref[:H, :], precision=_HI)
    h = h + jax.lax.dot(b, wn_ref[H:, :], precision=_HI)
    h = jnp.maximum(h + b_ref[...], 0.0)
    oa_ref[...] = h[:, :H]
    ob_ref[...] = h[:, H:]
    od_ref[...] = dinv.reshape(od_ref.shape)


_R0 = 1024


def _layer0(xp, aggdeg, degp, W0_self, W0_neigh, b0):
    return pl.pallas_call(
        _layer0_body,
        grid=(NPAD // _R0,),
        in_specs=[
            pl.BlockSpec((_R0, D), lambda i: (i, 0)),
            pl.BlockSpec((1, _R0, H), lambda i: (0, i, 0)),
            pl.BlockSpec((1, _R0, H), lambda i: (1, i, 0)),
            pl.BlockSpec((1, 16, _R0), lambda i: (0, 0, i)),
            pl.BlockSpec((D, D), lambda i: (0, 0)),
            pl.BlockSpec((D, D), lambda i: (0, 0)),
            pl.BlockSpec((1, D), lambda i: (0, 0)),
        ],
        out_specs=[
            pl.BlockSpec((_R0, H), lambda i: (i, 0)),
            pl.BlockSpec((_R0, H), lambda i: (i, 0)),
            pl.BlockSpec((_R0 // 128, 128), lambda i: (i, 0)),
        ],
        out_shape=[
            jax.ShapeDtypeStruct((NPAD, H), jnp.float32),
            jax.ShapeDtypeStruct((NPAD, H), jnp.float32),
            jax.ShapeDtypeStruct((NPAD // 128, 128), jnp.float32),
        ],
    )(xp, aggdeg, aggdeg, degp, W0_self, W0_neigh, b0)


def _layer1_body(at_ref, ht_ref, ws_ref, wn_ref, b_ref, o_ref):
    hl = ht_ref[0]
    hh = ht_ref[1]
    o = jax.lax.dot(hl, ws_ref[:H, :], precision=_HI)
    o = o + jax.lax.dot(hh, ws_ref[H:, :], precision=_HI)
    o = o + jax.lax.dot(at_ref[0], wn_ref[:H, :], precision=_HI)
    o = o + jax.lax.dot(at_ref[1], wn_ref[H:, :], precision=_HI)
    o_ref[...] = o + b_ref[...]


def _layer1(aggt, ht, W1_self, W1_neigh, b1):
    return pl.pallas_call(
        _layer1_body,
        out_shape=jax.ShapeDtypeStruct((NT, D), jnp.float32),
    )(aggt, ht, W1_self, W1_neigh, b1)


def kernel(x, edge_index, target_indices, W0_self, W0_neigh, b0,
           W1_self, W1_neigh, b1):
    f32 = jnp.float32
    x = x.astype(f32)
    src = edge_index[0]
    dst = edge_index[1]

    xp = jnp.zeros((NPAD, D), f32).at[:N_NODES].set(x)
    xa = xp[:, :H]
    xb = xp[:, H:]

    pad = EPAD - E
    srcp = jnp.concatenate([src, jnp.zeros((pad,), jnp.int32)]).reshape(
        EPAD // CHUNK, CHUNK)
    dstp = jnp.concatenate([dst, jnp.full((pad,), N_NODES, jnp.int32)]).reshape(
        EPAD // CHUNK, CHUNK)
    zrows = jnp.zeros((ROWS_PER_TILE, H), f32)
    zvec = jnp.zeros((NPAD,), f32)

    dummy = jnp.full((CAPT,), DUMMY, jnp.int32)

    agg_raw, degp = _seg0(xa, xb, srcp, dstp, zrows, zvec)
    srcf, dstf, cnt = _filt(srcp, dstp, target_indices, zvec, dummy)
    h1a, h1b, dinv2 = _layer0(xp, agg_raw, degp, W0_self, W0_neigh,
                              b0.reshape(1, D))
    aggt, ht = _seg1(h1a, h1b, srcf.reshape(16 * CAPC, CHUNK),
                     dstf.reshape(16 * CAPC, CHUNK), cnt, target_indices,
                     dinv2.reshape(NPAD), zrows)
    out = _layer1(aggt, ht, W1_self, W1_neigh, b1.reshape(1, D))
    return out


# speedup vs baseline: 1.3170x; 1.0060x over previous
import functools

import jax
import jax.numpy as jnp
from jax import lax
from jax.experimental import pallas as pl
from jax.experimental.pallas import tpu as pltpu
from jax.experimental.pallas import tpu_sc as plsc

N_NODES = 10000
NPAD = 10240
D = 256
H = 128
E = 160000
CHUNK = 64
GC = 8
NBUF = 4
EPAD = 163840
CHUNKS_PER_TILE = EPAD // 16 // CHUNK
GROUPS_PER_TILE = CHUNKS_PER_TILE // GC
EDGES_PER_GROUP = GC * CHUNK
ROWS_PER_TILE = NPAD // 16
NT = 1024
NTT = NT // CHUNK
TPT = CHUNK
DUMMY = N_NODES
EPG = GC * CHUNK
CAPT = 10752
CAPG = CAPT // EPG
CAPC = CAPT // CHUNK

NHALF = NPAD // 2
DUMLOC = NHALF
ACCROWS1 = 5248
RPT1 = ACCROWS1 // 16
CH1 = 64
EPG1 = GC * CH1
CAP1 = 10752
CAPC1 = CAP1 // CH1

_mesh = plsc.VectorSubcoreMesh(core_axis_name="c", subcore_axis_name="s")
_sc_params = pltpu.CompilerParams(needs_layout_passes=False)


def _edge_sweep(x_hbm, src_hbm, dst_hbm, zrows_hbm, chunk_base, n_groups,
                srcg, dstg, rows, acc, gsem, ssem, degloc=None):
    ones16 = jnp.ones((16,), jnp.float32)
    ch = dstg.shape[1]
    gc = dstg.shape[0]
    nbuf = len(rows)

    pltpu.sync_copy(src_hbm.at[pl.ds(chunk_base, gc)], srcg)
    pltpu.sync_copy(dst_hbm.at[pl.ds(chunk_base, gc)], dstg)
    for p in range(nbuf):
        pltpu.sync_copy(zrows_hbm.at[pl.ds(0, ch)], rows[p])
    for p in range(nbuf):
        pltpu.async_copy(rows[p], acc.at[dstg.at[0]], ssem[p], add=True)

    def group(g, carry):
        @pl.when(g > 0)
        def _stage():
            pltpu.sync_copy(src_hbm.at[pl.ds(chunk_base + g * gc, gc)], srcg)
            pltpu.sync_copy(dst_hbm.at[pl.ds(chunk_base + g * gc, gc)], dstg)
        for b in range(2):
            pltpu.make_async_copy(rows[b], acc.at[dstg.at[0]], ssem[b]).wait()
            pltpu.async_copy(x_hbm.at[srcg.at[b]], rows[b], gsem[b])
        for b in range(gc):
            p = b % nbuf
            pltpu.make_async_copy(x_hbm.at[srcg.at[b]], rows[p], gsem[p]).wait()
            if degloc is not None:
                for k in range(ch // 16):
                    idx16 = dstg[b, pl.ds(k * 16, 16)]
                    plsc.addupdate_scatter(degloc, [idx16], ones16)
            pltpu.async_copy(rows[p], acc.at[dstg.at[b]], ssem[p], add=True)
            if b + 2 < gc:
                q = (b + 2) % nbuf
                pltpu.make_async_copy(
                    rows[q], acc.at[dstg.at[0]], ssem[q]).wait()
                pltpu.async_copy(x_hbm.at[srcg.at[b + 2]], rows[q], gsem[q])
        return carry

    lax.fori_loop(0, n_groups, group, 0)
    for p in range(nbuf):
        pltpu.make_async_copy(rows[p], acc.at[dstg.at[0]], ssem[p]).wait()


def _filter_body(src_hbm, dst_hbm, ti_hbm, zvec_hbm, dummy_hbm,
                 srcf_hbm, dstf_hbm, cnt_hbm,
                 srcv, dstv, srcf, dstf, istgt, tivf, cntv):
    c = lax.axis_index("c")
    s = lax.axis_index("s")
    ones16 = jnp.ones((16,), jnp.float32)

    def run():
        pltpu.sync_copy(zvec_hbm, istgt)
        pltpu.sync_copy(ti_hbm, tivf)
        for g in range(NT // 16):
            t16 = tivf[pl.ds(g * 16, 16)]
            plsc.store_scatter(istgt, [t16], ones16)
        pltpu.sync_copy(src_hbm.at[pl.ds(s * CHUNKS_PER_TILE, CHUNKS_PER_TILE)],
                        srcv)
        pltpu.sync_copy(dst_hbm.at[pl.ds(s * CHUNKS_PER_TILE, CHUNKS_PER_TILE)],
                        dstv)
        pltpu.sync_copy(dummy_hbm, srcf)
        pltpu.sync_copy(dummy_hbm, dstf)

        def chunk(ch, off):
            for k in range(CHUNK // 16):
                d16 = dstv[ch, pl.ds(k * 16, 16)]
                s16 = srcv[ch, pl.ds(k * 16, 16)]
                m16 = plsc.load_gather(istgt, [d16]) > 0.5
                plsc.store_compressed(srcf.at[pl.ds(off, 16)], s16, mask=m16)
                plsc.store_compressed(dstf.at[pl.ds(off, 16)], d16, mask=m16)
                off = off + plsc.all_reduce_population_count(m16)[0]
            return off

        off = lax.fori_loop(0, CHUNKS_PER_TILE, chunk, jnp.int32(0))
        off_r = ((off + EPG - 1) // EPG) * EPG
        pltpu.sync_copy(srcf, srcf_hbm.at[s])
        pltpu.sync_copy(dstf, dstf_hbm.at[s])
        cntv[...] = jnp.full((16,), 0, jnp.int32) + off_r
        pltpu.sync_copy(cntv, cnt_hbm.at[s])

    pl.when(c == 0)(run)


_filt = functools.partial(
    pl.kernel,
    mesh=_mesh,
    out_type=[
        jax.ShapeDtypeStruct((16, CAPT), jnp.int32),
        jax.ShapeDtypeStruct((16, CAPT), jnp.int32),
        jax.ShapeDtypeStruct((16, 16), jnp.int32),
    ],
    scratch_types=[
        pltpu.VMEM((CHUNKS_PER_TILE, CHUNK), jnp.int32),
        pltpu.VMEM((CHUNKS_PER_TILE, CHUNK), jnp.int32),
        pltpu.VMEM((CAPT,), jnp.int32),
        pltpu.VMEM((CAPT,), jnp.int32),
        pltpu.VMEM((NPAD,), jnp.float32),
        pltpu.VMEM((NT,), jnp.int32),
        pltpu.VMEM((16,), jnp.int32),
    ],
    compiler_params=_sc_params,
)(_filter_body)


def _segsum0_body(xa_hbm, xb_hbm, src_hbm, dst_hbm, zrows_hbm, zvec_hbm,
                  agg_hbm, degp_hbm, srcg, dstg,
                  rows0, rows1, rows2, rows3, degloc, acc,
                  gsem0, gsem1, gsem2, gsem3, ssem0, ssem1, ssem2, ssem3):
    c = lax.axis_index("c")
    s = lax.axis_index("s")
    rows = (rows0, rows1, rows2, rows3)
    gsem = (gsem0, gsem1, gsem2, gsem3)
    ssem = (ssem0, ssem1, ssem2, ssem3)
    pltpu.sync_copy(zrows_hbm, acc.at[pl.ds(s * ROWS_PER_TILE, ROWS_PER_TILE)])
    pltpu.sync_copy(zvec_hbm, degloc)
    plsc.subcore_barrier()

    pl.when(c == 0)(lambda: _edge_sweep(
        xa_hbm, src_hbm, dst_hbm, zrows_hbm, s * CHUNKS_PER_TILE,
        GROUPS_PER_TILE, srcg, dstg, rows, acc, gsem, ssem, degloc))
    pl.when(c == 1)(lambda: _edge_sweep(
        xb_hbm, src_hbm, dst_hbm, zrows_hbm, s * CHUNKS_PER_TILE,
        GROUPS_PER_TILE, srcg, dstg, rows, acc, gsem, ssem, degloc))
    pltpu.sync_copy(degloc, degp_hbm.at[c, s])
    plsc.subcore_barrier()
    pltpu.sync_copy(acc.at[pl.ds(s * ROWS_PER_TILE, ROWS_PER_TILE)],
                    agg_hbm.at[c, pl.ds(s * ROWS_PER_TILE, ROWS_PER_TILE)])


_seg0 = functools.partial(
    pl.kernel,
    mesh=_mesh,
    out_type=[
        jax.ShapeDtypeStruct((2, NPAD, H), jnp.float32),
        jax.ShapeDtypeStruct((2, 16, NPAD), jnp.float32),
    ],
    scratch_types=[
        pltpu.VMEM((GC, CHUNK), jnp.int32),
        pltpu.VMEM((GC, CHUNK), jnp.int32),
        pltpu.VMEM((CHUNK, H), jnp.float32),
        pltpu.VMEM((CHUNK, H), jnp.float32),
        pltpu.VMEM((CHUNK, H), jnp.float32),
        pltpu.VMEM((CHUNK, H), jnp.float32),
        pltpu.VMEM((NPAD,), jnp.float32),
        pltpu.VMEM_SHARED((NPAD, H), jnp.float32),
        pltpu.SemaphoreType.DMA,
        pltpu.SemaphoreType.DMA,
        pltpu.SemaphoreType.DMA,
        pltpu.SemaphoreType.DMA,
        pltpu.SemaphoreType.DMA,
        pltpu.SemaphoreType.DMA,
        pltpu.SemaphoreType.DMA,
        pltpu.SemaphoreType.DMA,
    ],
    compiler_params=_sc_params,
)(_segsum0_body)


def _segsum1_body(ha_hbm, hb_hbm, src_hbm, dst_hbm, cnt_hbm, ti_hbm, dinv_hbm,
                  zrows_hbm, aggt_hbm, ht_hbm,
                  srcg, dstg, tiv, cntv, rows0, rows1, rows2, rows3, dinvv,
                  acc, gsem0, gsem1, gsem2, gsem3, ssem0, ssem1, ssem2, ssem3):
    c = lax.axis_index("c")
    s = lax.axis_index("s")
    rows = (rows0, rows1, rows2, rows3)
    gsem = (gsem0, gsem1, gsem2, gsem3)
    ssem = (ssem0, ssem1, ssem2, ssem3)
    def zinit():
        pltpu.sync_copy(ti_hbm.at[pl.ds(s * TPT, TPT)], tiv)
        pltpu.sync_copy(zrows_hbm.at[pl.ds(0, TPT)], rows0)
        pltpu.sync_copy(rows0, acc.at[tiv])
    pl.when(s < NTT)(zinit)
    pltpu.sync_copy(dinv_hbm, dinvv)
    pltpu.sync_copy(cnt_hbm.at[s], cntv)
    n_groups = cntv[...][0] // EPG
    plsc.subcore_barrier()

    def run(h_hbm):
        _edge_sweep(h_hbm, src_hbm, dst_hbm, zrows_hbm, s * CAPC, n_groups,
                    srcg, dstg, rows, acc, gsem, ssem)
        pl.when(s < NTT)(
            lambda: pltpu.async_copy(h_hbm.at[tiv], rows1, gsem1).wait())

    pl.when(c == 0)(lambda: run(ha_hbm))
    pl.when(c == 1)(lambda: run(hb_hbm))
    plsc.subcore_barrier()

    def targets():
        pltpu.async_copy(acc.at[tiv], rows0, gsem0).wait()
        for g in range(TPT // 16):
            tiv16 = tiv[pl.ds(g * 16, 16)]
            dinv16 = plsc.load_gather(dinvv, [tiv16])
            for l in range(16):
                d = dinv16[l]
                r = g * 16 + l
                for k in range(H // 16):
                    rows0[r, pl.ds(k * 16, 16)] = (
                        rows0[r, pl.ds(k * 16, 16)] * d)
        pltpu.sync_copy(rows0, aggt_hbm.at[c, pl.ds(s * TPT, TPT)])
        pltpu.sync_copy(rows1, ht_hbm.at[c, pl.ds(s * TPT, TPT)])

    pl.when(s < NTT)(targets)


_seg1 = functools.partial(
    pl.kernel,
    mesh=_mesh,
    out_type=[
        jax.ShapeDtypeStruct((2, NT, H), jnp.float32),
        jax.ShapeDtypeStruct((2, NT, H), jnp.float32),
    ],
    scratch_types=[
        pltpu.VMEM((GC, CHUNK), jnp.int32),
        pltpu.VMEM((GC, CHUNK), jnp.int32),
        pltpu.VMEM((TPT,), jnp.int32),
        pltpu.VMEM((16,), jnp.int32),
        pltpu.VMEM((CHUNK, H), jnp.float32),
        pltpu.VMEM((CHUNK, H), jnp.float32),
        pltpu.VMEM((CHUNK, H), jnp.float32),
        pltpu.VMEM((CHUNK, H), jnp.float32),
        pltpu.VMEM((NPAD,), jnp.float32),
        pltpu.VMEM_SHARED((NPAD, H), jnp.float32),
        pltpu.SemaphoreType.DMA,
        pltpu.SemaphoreType.DMA,
        pltpu.SemaphoreType.DMA,
        pltpu.SemaphoreType.DMA,
        pltpu.SemaphoreType.DMA,
        pltpu.SemaphoreType.DMA,
        pltpu.SemaphoreType.DMA,
        pltpu.SemaphoreType.DMA,
    ],
    compiler_params=_sc_params,
)(_segsum1_body)


_HI = jax.lax.Precision.HIGHEST


def _layer0_body(x_ref, aa_ref, ab_ref, dp_ref, ws_ref, wn_ref, b_ref,
                 oa_ref, ob_ref, od_ref):
    deg = jnp.sum(dp_ref[0], axis=0)
    dinv = 1.0 / jnp.maximum(deg, 1.0)
    dcol = dinv[:, None]
    a = aa_ref[0] * dcol
    b = ab_ref[0] * dcol
    h = jax.lax.dot(x_ref[...], ws_ref[...], precision=_HI)
    h = h + jax.lax.dot(a, wn_ref[:H, :], precision=_HI)
    h = h + jax.lax.dot(b, wn_ref[H:, :], precision=_HI)
    h = jnp.maximum(h + b_ref[...], 0.0)
    oa_ref[...] = h[:, :H]
    ob_ref[...] = h[:, H:]
    od_ref[...] = dinv.reshape(od_ref.shape)


_R0 = 1024


def _layer0(xp, aggdeg, degp, W0_self, W0_neigh, b0):
    return pl.pallas_call(
        _layer0_body,
        grid=(NPAD // _R0,),
        in_specs=[
            pl.BlockSpec((_R0, D), lambda i: (i, 0)),
            pl.BlockSpec((1, _R0, H), lambda i: (0, i, 0)),
            pl.BlockSpec((1, _R0, H), lambda i: (1, i, 0)),
            pl.BlockSpec((1, 16, _R0), lambda i: (0, 0, i)),
            pl.BlockSpec((D, D), lambda i: (0, 0)),
            pl.BlockSpec((D, D), lambda i: (0, 0)),
            pl.BlockSpec((1, D), lambda i: (0, 0)),
        ],
        out_specs=[
            pl.BlockSpec((_R0, H), lambda i: (i, 0)),
            pl.BlockSpec((_R0, H), lambda i: (i, 0)),
            pl.BlockSpec((_R0 // 128, 128), lambda i: (i, 0)),
        ],
        out_shape=[
            jax.ShapeDtypeStruct((NPAD, H), jnp.float32),
            jax.ShapeDtypeStruct((NPAD, H), jnp.float32),
            jax.ShapeDtypeStruct((NPAD // 128, 128), jnp.float32),
        ],
    )(xp, aggdeg, aggdeg, degp, W0_self, W0_neigh, b0)


def _layer1_body(at_ref, ht_ref, ws_ref, wn_ref, b_ref, o_ref):
    hl = ht_ref[0]
    hh = ht_ref[1]
    o = jax.lax.dot(hl, ws_ref[:H, :], precision=_HI)
    o = o + jax.lax.dot(hh, ws_ref[H:, :], precision=_HI)
    o = o + jax.lax.dot(at_ref[0], wn_ref[:H, :], precision=_HI)
    o = o + jax.lax.dot(at_ref[1], wn_ref[H:, :], precision=_HI)
    o_ref[...] = o + b_ref[...]


def _layer1(aggt, ht, W1_self, W1_neigh, b1):
    return pl.pallas_call(
        _layer1_body,
        out_shape=jax.ShapeDtypeStruct((NT, D), jnp.float32),
    )(aggt, ht, W1_self, W1_neigh, b1)


def kernel(x, edge_index, target_indices, W0_self, W0_neigh, b0,
           W1_self, W1_neigh, b1):
    f32 = jnp.float32
    x = x.astype(f32)
    src = edge_index[0]
    dst = edge_index[1]

    xp = jnp.zeros((NPAD, D), f32).at[:N_NODES].set(x)
    xa = xp[:, :H]
    xb = xp[:, H:]

    pad = EPAD - E
    srcp = jnp.concatenate([src, jnp.zeros((pad,), jnp.int32)]).reshape(
        EPAD // CHUNK, CHUNK)
    dstp = jnp.concatenate([dst, jnp.full((pad,), N_NODES, jnp.int32)]).reshape(
        EPAD // CHUNK, CHUNK)
    zrows = jnp.zeros((ROWS_PER_TILE, H), f32)
    zvec = jnp.zeros((NPAD,), f32)

    dummy = jnp.full((CAPT,), DUMMY, jnp.int32)

    srcf, dstf, cnt = _filt(srcp, dstp, target_indices, zvec, dummy)
    agg_raw, degp = _seg0(xa, xb, srcp, dstp, zrows, zvec)
    h1a, h1b, dinv2 = _layer0(xp, agg_raw, degp, W0_self, W0_neigh,
                              b0.reshape(1, D))
    aggt, ht = _seg1(h1a, h1b, srcf.reshape(16 * CAPC, CHUNK),
                     dstf.reshape(16 * CAPC, CHUNK), cnt, target_indices,
                     dinv2.reshape(NPAD), zrows)
    out = _layer1(aggt, ht, W1_self, W1_neigh, b1.reshape(1, D))
    return out


# speedup vs baseline: 1.3976x; 1.0612x over previous
import functools

import jax
import jax.numpy as jnp
from jax import lax
from jax.experimental import pallas as pl
from jax.experimental.pallas import tpu as pltpu
from jax.experimental.pallas import tpu_sc as plsc

N_NODES = 10000
NPAD = 10240
D = 256
H = 128
E = 160000
CHUNK = 64
GC = 8
GC1 = 16
NBUF = 4
EPAD = 163840
CHUNKS_PER_TILE = EPAD // 16 // CHUNK
GROUPS_PER_TILE = CHUNKS_PER_TILE // GC1
EDGES_PER_GROUP = GC * CHUNK
ROWS_PER_TILE = NPAD // 16
NT = 1024
CH2 = 128
GC2 = 4
NTT = NT // CH2
TPT = CH2
DUMMY = N_NODES
EPG = GC * CHUNK
CAPT = 10752
CAPG = CAPT // EPG
CAPC = CAPT // CH2

NHALF = NPAD // 2
DUMLOC = NHALF
ACCROWS1 = 5248
RPT1 = ACCROWS1 // 16
CH1 = 64
EPG1 = GC * CH1
CAP1 = 10752
CAPC1 = CAP1 // CH1

_mesh = plsc.VectorSubcoreMesh(core_axis_name="c", subcore_axis_name="s")
_sc_params = pltpu.CompilerParams(needs_layout_passes=False)


def _edge_sweep(x_hbm, src_hbm, dst_hbm, zrows_hbm, chunk_base, n_groups,
                srcg, dstg, rows, acc, gsem, ssem, degloc=None):
    ones16 = jnp.ones((16,), jnp.float32)
    ch = dstg.shape[1]
    gc = dstg.shape[0]
    nbuf = len(rows)

    pltpu.sync_copy(src_hbm.at[pl.ds(chunk_base, gc)], srcg)
    pltpu.sync_copy(dst_hbm.at[pl.ds(chunk_base, gc)], dstg)
    for p in range(nbuf):
        pltpu.sync_copy(zrows_hbm.at[pl.ds(0, ch)], rows[p])
    for p in range(nbuf):
        pltpu.async_copy(rows[p], acc.at[dstg.at[0]], ssem[p], add=True)

    def group(g, carry):
        @pl.when(g > 0)
        def _stage():
            pltpu.sync_copy(src_hbm.at[pl.ds(chunk_base + g * gc, gc)], srcg)
            pltpu.sync_copy(dst_hbm.at[pl.ds(chunk_base + g * gc, gc)], dstg)
        for b in range(2):
            pltpu.make_async_copy(rows[b], acc.at[dstg.at[0]], ssem[b]).wait()
            pltpu.async_copy(x_hbm.at[srcg.at[b]], rows[b], gsem[b])
        for b in range(gc):
            p = b % nbuf
            pltpu.make_async_copy(x_hbm.at[srcg.at[b]], rows[p], gsem[p]).wait()
            if degloc is not None:
                for k in range(ch // 16):
                    idx16 = dstg[b, pl.ds(k * 16, 16)]
                    plsc.addupdate_scatter(degloc, [idx16], ones16)
            pltpu.async_copy(rows[p], acc.at[dstg.at[b]], ssem[p], add=True)
            if b + 2 < gc:
                q = (b + 2) % nbuf
                pltpu.make_async_copy(
                    rows[q], acc.at[dstg.at[0]], ssem[q]).wait()
                pltpu.async_copy(x_hbm.at[srcg.at[b + 2]], rows[q], gsem[q])
        return carry

    lax.fori_loop(0, n_groups, group, 0)
    for p in range(nbuf):
        pltpu.make_async_copy(rows[p], acc.at[dstg.at[0]], ssem[p]).wait()


def _filter_body(src_hbm, dst_hbm, ti_hbm, zvec_hbm, dummy_hbm,
                 srcf_hbm, dstf_hbm, cnt_hbm,
                 srcv, dstv, srcf, dstf, istgt, tivf, cntv):
    c = lax.axis_index("c")
    s = lax.axis_index("s")
    ones16 = jnp.ones((16,), jnp.float32)

    def run():
        pltpu.sync_copy(zvec_hbm, istgt)
        pltpu.sync_copy(ti_hbm, tivf)
        for g in range(NT // 16):
            t16 = tivf[pl.ds(g * 16, 16)]
            plsc.store_scatter(istgt, [t16], ones16)
        pltpu.sync_copy(src_hbm.at[pl.ds(s * CHUNKS_PER_TILE, CHUNKS_PER_TILE)],
                        srcv)
        pltpu.sync_copy(dst_hbm.at[pl.ds(s * CHUNKS_PER_TILE, CHUNKS_PER_TILE)],
                        dstv)
        pltpu.sync_copy(dummy_hbm, srcf)
        pltpu.sync_copy(dummy_hbm, dstf)

        def chunk(ch, off):
            for k in range(CHUNK // 16):
                d16 = dstv[ch, pl.ds(k * 16, 16)]
                s16 = srcv[ch, pl.ds(k * 16, 16)]
                m16 = plsc.load_gather(istgt, [d16]) > 0.5
                plsc.store_compressed(srcf.at[pl.ds(off, 16)], s16, mask=m16)
                plsc.store_compressed(dstf.at[pl.ds(off, 16)], d16, mask=m16)
                off = off + plsc.all_reduce_population_count(m16)[0]
            return off

        off = lax.fori_loop(0, CHUNKS_PER_TILE, chunk, jnp.int32(0))
        off_r = ((off + EPG - 1) // EPG) * EPG
        pltpu.sync_copy(srcf, srcf_hbm.at[s])
        pltpu.sync_copy(dstf, dstf_hbm.at[s])
        cntv[...] = jnp.full((16,), 0, jnp.int32) + off_r
        pltpu.sync_copy(cntv, cnt_hbm.at[s])

    pl.when(c == 0)(run)


_filt = functools.partial(
    pl.kernel,
    mesh=_mesh,
    out_type=[
        jax.ShapeDtypeStruct((16, CAPT), jnp.int32),
        jax.ShapeDtypeStruct((16, CAPT), jnp.int32),
        jax.ShapeDtypeStruct((16, 16), jnp.int32),
    ],
    scratch_types=[
        pltpu.VMEM((CHUNKS_PER_TILE, CHUNK), jnp.int32),
        pltpu.VMEM((CHUNKS_PER_TILE, CHUNK), jnp.int32),
        pltpu.VMEM((CAPT,), jnp.int32),
        pltpu.VMEM((CAPT,), jnp.int32),
        pltpu.VMEM((NPAD,), jnp.float32),
        pltpu.VMEM((NT,), jnp.int32),
        pltpu.VMEM((16,), jnp.int32),
    ],
    compiler_params=_sc_params,
)(_filter_body)


def _segsum0_body(xa_hbm, xb_hbm, src_hbm, dst_hbm, zrows_hbm, zvec_hbm,
                  agg_hbm, degp_hbm, srcg, dstg,
                  rows0, rows1, rows2, rows3, degloc, acc,
                  gsem0, gsem1, gsem2, gsem3, ssem0, ssem1, ssem2, ssem3):
    c = lax.axis_index("c")
    s = lax.axis_index("s")
    rows = (rows0, rows1, rows2, rows3)
    gsem = (gsem0, gsem1, gsem2, gsem3)
    ssem = (ssem0, ssem1, ssem2, ssem3)
    pltpu.sync_copy(zrows_hbm, acc.at[pl.ds(s * ROWS_PER_TILE, ROWS_PER_TILE)])
    pltpu.sync_copy(zvec_hbm, degloc)
    plsc.subcore_barrier()

    pl.when(c == 0)(lambda: _edge_sweep(
        xa_hbm, src_hbm, dst_hbm, zrows_hbm, s * CHUNKS_PER_TILE,
        GROUPS_PER_TILE, srcg, dstg, rows, acc, gsem, ssem, degloc))
    pl.when(c == 1)(lambda: _edge_sweep(
        xb_hbm, src_hbm, dst_hbm, zrows_hbm, s * CHUNKS_PER_TILE,
        GROUPS_PER_TILE, srcg, dstg, rows, acc, gsem, ssem, degloc))
    pltpu.sync_copy(degloc, degp_hbm.at[c, s])
    plsc.subcore_barrier()
    pltpu.sync_copy(acc.at[pl.ds(s * ROWS_PER_TILE, ROWS_PER_TILE)],
                    agg_hbm.at[c, pl.ds(s * ROWS_PER_TILE, ROWS_PER_TILE)])


_seg0 = functools.partial(
    pl.kernel,
    mesh=_mesh,
    out_type=[
        jax.ShapeDtypeStruct((2, NPAD, H), jnp.float32),
        jax.ShapeDtypeStruct((2, 16, NPAD), jnp.float32),
    ],
    scratch_types=[
        pltpu.VMEM((GC1, CHUNK), jnp.int32),
        pltpu.VMEM((GC1, CHUNK), jnp.int32),
        pltpu.VMEM((CHUNK, H), jnp.float32),
        pltpu.VMEM((CHUNK, H), jnp.float32),
        pltpu.VMEM((CHUNK, H), jnp.float32),
        pltpu.VMEM((CHUNK, H), jnp.float32),
        pltpu.VMEM((NPAD,), jnp.float32),
        pltpu.VMEM_SHARED((NPAD, H), jnp.float32),
        pltpu.SemaphoreType.DMA,
        pltpu.SemaphoreType.DMA,
        pltpu.SemaphoreType.DMA,
        pltpu.SemaphoreType.DMA,
        pltpu.SemaphoreType.DMA,
        pltpu.SemaphoreType.DMA,
        pltpu.SemaphoreType.DMA,
        pltpu.SemaphoreType.DMA,
    ],
    compiler_params=_sc_params,
)(_segsum0_body)


def _segsum1_body(ha_hbm, hb_hbm, src_hbm, dst_hbm, cnt_hbm, ti_hbm, dinv_hbm,
                  zrows_hbm, aggt_hbm, ht_hbm,
                  srcg, dstg, tiv, cntv, rows0, rows1, dinvv,
                  acc, gsem0, gsem1, ssem0, ssem1):
    c = lax.axis_index("c")
    s = lax.axis_index("s")
    rows = (rows0, rows1)
    gsem = (gsem0, gsem1)
    ssem = (ssem0, ssem1)
    def zinit():
        pltpu.sync_copy(ti_hbm.at[pl.ds(s * TPT, TPT)], tiv)
        pltpu.sync_copy(zrows_hbm.at[pl.ds(0, TPT)], rows0)
        pltpu.sync_copy(rows0, acc.at[tiv])
    pl.when(s < NTT)(zinit)
    pltpu.sync_copy(dinv_hbm, dinvv)
    pltpu.sync_copy(cnt_hbm.at[s], cntv)
    n_groups = cntv[...][0] // EPG
    plsc.subcore_barrier()

    def run(h_hbm):
        _edge_sweep(h_hbm, src_hbm, dst_hbm, zrows_hbm, s * CAPC, n_groups,
                    srcg, dstg, rows, acc, gsem, ssem)
        pl.when(s < NTT)(
            lambda: pltpu.async_copy(h_hbm.at[tiv], rows1, gsem1).wait())

    pl.when(c == 0)(lambda: run(ha_hbm))
    pl.when(c == 1)(lambda: run(hb_hbm))
    plsc.subcore_barrier()

    def targets():
        pltpu.async_copy(acc.at[tiv], rows0, gsem0).wait()
        for g in range(TPT // 16):
            tiv16 = tiv[pl.ds(g * 16, 16)]
            dinv16 = plsc.load_gather(dinvv, [tiv16])
            for l in range(16):
                d = dinv16[l]
                r = g * 16 + l
                for k in range(H // 16):
                    rows0[r, pl.ds(k * 16, 16)] = (
                        rows0[r, pl.ds(k * 16, 16)] * d)
        pltpu.sync_copy(rows0, aggt_hbm.at[c, pl.ds(s * TPT, TPT)])
        pltpu.sync_copy(rows1, ht_hbm.at[c, pl.ds(s * TPT, TPT)])

    pl.when(s < NTT)(targets)


_seg1 = functools.partial(
    pl.kernel,
    mesh=_mesh,
    out_type=[
        jax.ShapeDtypeStruct((2, NT, H), jnp.float32),
        jax.ShapeDtypeStruct((2, NT, H), jnp.float32),
    ],
    scratch_types=[
        pltpu.VMEM((GC2, CH2), jnp.int32),
        pltpu.VMEM((GC2, CH2), jnp.int32),
        pltpu.VMEM((TPT,), jnp.int32),
        pltpu.VMEM((16,), jnp.int32),
        pltpu.VMEM((CH2, H), jnp.float32),
        pltpu.VMEM((CH2, H), jnp.float32),
        pltpu.VMEM((NPAD,), jnp.float32),
        pltpu.VMEM_SHARED((NPAD, H), jnp.float32),
        pltpu.SemaphoreType.DMA,
        pltpu.SemaphoreType.DMA,
        pltpu.SemaphoreType.DMA,
        pltpu.SemaphoreType.DMA,
    ],
    compiler_params=_sc_params,
)(_segsum1_body)


_HI = jax.lax.Precision.HIGHEST


def _layer0_body(x_ref, aa_ref, ab_ref, dp_ref, ws_ref, wn_ref, b_ref,
                 oa_ref, ob_ref, od_ref):
    deg = jnp.sum(dp_ref[0], axis=0)
    dinv = 1.0 / jnp.maximum(deg, 1.0)
    dcol = dinv[:, None]
    a = aa_ref[0] * dcol
    b = ab_ref[0] * dcol
    h = jax.lax.dot(x_ref[...], ws_ref[...], precision=_HI)
    h = h + jax.lax.dot(a, wn_ref[:H, :], precision=_HI)
    h = h + jax.lax.dot(b, wn_ref[H:, :], precision=_HI)
    h = jnp.maximum(h + b_ref[...], 0.0)
    oa_ref[...] = h[:, :H]
    ob_ref[...] = h[:, H:]
    od_ref[...] = dinv.reshape(od_ref.shape)


_R0 = 1024


def _layer0(xp, aggdeg, degp, W0_self, W0_neigh, b0):
    return pl.pallas_call(
        _layer0_body,
        grid=(NPAD // _R0,),
        in_specs=[
            pl.BlockSpec((_R0, D), lambda i: (i, 0)),
            pl.BlockSpec((1, _R0, H), lambda i: (0, i, 0)),
            pl.BlockSpec((1, _R0, H), lambda i: (1, i, 0)),
            pl.BlockSpec((1, 16, _R0), lambda i: (0, 0, i)),
            pl.BlockSpec((D, D), lambda i: (0, 0)),
            pl.BlockSpec((D, D), lambda i: (0, 0)),
            pl.BlockSpec((1, D), lambda i: (0, 0)),
        ],
        out_specs=[
            pl.BlockSpec((_R0, H), lambda i: (i, 0)),
            pl.BlockSpec((_R0, H), lambda i: (i, 0)),
            pl.BlockSpec((_R0 // 128, 128), lambda i: (i, 0)),
        ],
        out_shape=[
            jax.ShapeDtypeStruct((NPAD, H), jnp.float32),
            jax.ShapeDtypeStruct((NPAD, H), jnp.float32),
            jax.ShapeDtypeStruct((NPAD // 128, 128), jnp.float32),
        ],
    )(xp, aggdeg, aggdeg, degp, W0_self, W0_neigh, b0)


def _layer1_body(at_ref, ht_ref, ws_ref, wn_ref, b_ref, o_ref):
    hl = ht_ref[0]
    hh = ht_ref[1]
    o = jax.lax.dot(hl, ws_ref[:H, :], precision=_HI)
    o = o + jax.lax.dot(hh, ws_ref[H:, :], precision=_HI)
    o = o + jax.lax.dot(at_ref[0], wn_ref[:H, :], precision=_HI)
    o = o + jax.lax.dot(at_ref[1], wn_ref[H:, :], precision=_HI)
    o_ref[...] = o + b_ref[...]


def _layer1(aggt, ht, W1_self, W1_neigh, b1):
    return pl.pallas_call(
        _layer1_body,
        out_shape=jax.ShapeDtypeStruct((NT, D), jnp.float32),
    )(aggt, ht, W1_self, W1_neigh, b1)


def kernel(x, edge_index, target_indices, W0_self, W0_neigh, b0,
           W1_self, W1_neigh, b1):
    f32 = jnp.float32
    x = x.astype(f32)
    src = edge_index[0]
    dst = edge_index[1]

    xp = jnp.zeros((NPAD, D), f32).at[:N_NODES].set(x)
    xa = xp[:, :H]
    xb = xp[:, H:]

    pad = EPAD - E
    srcp = jnp.concatenate([src, jnp.zeros((pad,), jnp.int32)]).reshape(
        EPAD // CHUNK, CHUNK)
    dstp = jnp.concatenate([dst, jnp.full((pad,), N_NODES, jnp.int32)]).reshape(
        EPAD // CHUNK, CHUNK)
    zrows = jnp.zeros((ROWS_PER_TILE, H), f32)
    zvec = jnp.zeros((NPAD,), f32)

    dummy = jnp.full((CAPT,), DUMMY, jnp.int32)

    srcf, dstf, cnt = _filt(srcp, dstp, target_indices, zvec, dummy)
    agg_raw, degp = _seg0(xa, xb, srcp, dstp, zrows, zvec)
    h1a, h1b, dinv2 = _layer0(xp, agg_raw, degp, W0_self, W0_neigh,
                              b0.reshape(1, D))
    aggt, ht = _seg1(h1a, h1b, srcf.reshape(16 * CAPC, CH2),
                     dstf.reshape(16 * CAPC, CH2), cnt, target_indices,
                     dinv2.reshape(NPAD), zrows)
    out = _layer1(aggt, ht, W1_self, W1_neigh, b1.reshape(1, D))
    return out
